# Initial kernel scaffold; baseline (speedup 1.0000x reference)
import jax
import jax.numpy as jnp
from jax.experimental import pallas as pl


def kernel(h, edge_index, e, W_emb, b_emb, sage_W, sage_b, bn_gamma, bn_beta, dec1_W0, dec1_b0, dec1_W1, dec1_b1, dec1_W2, dec1_b2, dec2_W, dec2_b, mlp_W0, mlp_b0, mlp_W1, mlp_b1, mlp_W2, mlp_b2):
    raise NotImplementedError("write your pallas kernel here")


# speedup vs baseline: 14.1353x; 14.1353x over previous
import functools

import jax
import jax.numpy as jnp
from jax.experimental import pallas as pl

N_GRAPHS = 100
N_PER = 100
EPG = 1600
HID = 128


def _sage_body(h_ref, src_ref, dst_ref, W_emb_ref, b_emb_ref, sage_W_ref,
               sage_b_ref, bn_gamma_ref, bn_beta_ref, mlp_W0_ref, mlp_b0_ref,
               mlp_W1_ref, mlp_b1_ref, mlp_W2_ref, mlp_b2_ref, out_ref):
    g = pl.program_id(0)
    off = g * N_PER

    src = src_ref[0] - off
    dst = dst_ref[0] - off

    rows = jax.lax.broadcasted_iota(jnp.int32, (N_PER, EPG), 0)
    oh_dst_t = (rows == dst).astype(jnp.float32)
    oh_src_t = (rows == src).astype(jnp.float32)
    A = jax.lax.dot_general(
        oh_dst_t, oh_src_t, (((1,), (1,)), ((), ())),
        preferred_element_type=jnp.float32)
    deg = jnp.maximum(jnp.sum(A, axis=1, keepdims=True), 1.0)

    x = jnp.dot(h_ref[0], W_emb_ref[...],
                preferred_element_type=jnp.float32) + b_emb_ref[...]

    inv_bn = 1.0 / jnp.sqrt(1.0 + 1e-5)
    for i in range(4):
        agg = jnp.dot(A, x, preferred_element_type=jnp.float32) / deg
        w_top = sage_W_ref[i, :HID, :]
        w_bot = sage_W_ref[i, HID:, :]
        bundle = (jnp.dot(x, w_top, preferred_element_type=jnp.float32)
                  + jnp.dot(agg, w_bot, preferred_element_type=jnp.float32)
                  + sage_b_ref[i:i + 1, :])
        nrm = jnp.sqrt(jnp.sum(bundle * bundle, axis=1, keepdims=True))
        bundle = bundle / jnp.maximum(nrm, 1e-12)
        xx = jnp.maximum(bundle, 0.0)
        xx = bn_gamma_ref[i:i + 1, :] * (xx * inv_bn) + bn_beta_ref[i:i + 1, :]
        x = x + xx

    hg = jnp.mean(x, axis=0, keepdims=True)
    y = jnp.maximum(jnp.dot(hg, mlp_W0_ref[...],
                            preferred_element_type=jnp.float32)
                    + mlp_b0_ref[...], 0.0)
    y = jnp.maximum(jnp.dot(y, mlp_W1_ref[...],
                            preferred_element_type=jnp.float32)
                    + mlp_b1_ref[...], 0.0)
    z = jnp.sum(y * mlp_W2_ref[...], axis=1, keepdims=True) + mlp_b2_ref[...]
    out_ref[...] = jax.nn.sigmoid(z).reshape(1, 1, 1)


def kernel(h, edge_index, e, W_emb, b_emb, sage_W, sage_b, bn_gamma, bn_beta,
           dec1_W0, dec1_b0, dec1_W1, dec1_b1, dec1_W2, dec1_b2, dec2_W,
           dec2_b, mlp_W0, mlp_b0, mlp_W1, mlp_b1, mlp_W2, mlp_b2):
    h3 = h.reshape(N_GRAPHS, N_PER, HID)
    srcm = edge_index[0].reshape(N_GRAPHS, 1, EPG)
    dstm = edge_index[1].reshape(N_GRAPHS, 1, EPG)

    full = lambda *shape: pl.BlockSpec(shape, lambda g: (0,) * len(shape))
    grid_spec = pl.GridSpec(
        grid=(N_GRAPHS,),
        in_specs=[
            pl.BlockSpec((1, N_PER, HID), lambda g: (g, 0, 0)),
            pl.BlockSpec((1, 1, EPG), lambda g: (g, 0, 0)),
            pl.BlockSpec((1, 1, EPG), lambda g: (g, 0, 0)),
            full(HID, HID),
            full(1, HID),
            full(4, 2 * HID, HID),
            full(4, HID),
            full(4, HID),
            full(4, HID),
            full(HID, HID // 2),
            full(1, HID // 2),
            full(HID // 2, HID // 4),
            full(1, HID // 4),
            full(1, HID // 4),
            full(1, 1),
        ],
        out_specs=pl.BlockSpec((1, 1, 1), lambda g: (g, 0, 0)),
    )
    out = pl.pallas_call(
        _sage_body,
        grid_spec=grid_spec,
        out_shape=jax.ShapeDtypeStruct((N_GRAPHS, 1, 1), jnp.float32),
    )(h3, srcm, dstm, W_emb, b_emb.reshape(1, HID), sage_W, sage_b,
      bn_gamma, bn_beta, mlp_W0, mlp_b0.reshape(1, HID // 2), mlp_W1,
      mlp_b1.reshape(1, HID // 4), mlp_W2.reshape(1, HID // 4),
      mlp_b2.reshape(1, 1))
    return out.reshape(N_GRAPHS)


# speedup vs baseline: 39.2250x; 2.7750x over previous
import jax
import jax.numpy as jnp
from jax.experimental import pallas as pl

N_GRAPHS = 100
N_PER = 100
EPG = 1600
HID = 128
G_PER = 4


def _sage_body(h_ref, src_ref, dst_ref, W_emb_ref, b_emb_ref, sage_W_ref,
               sage_b_ref, bn_gamma_ref, bn_beta_ref, mlp_W0_ref, mlp_b0_ref,
               mlp_W1_ref, mlp_b1_ref, mlp_W2_ref, mlp_b2_ref, out_ref):
    g0 = pl.program_id(0) * G_PER
    rows = jax.lax.broadcasted_iota(jnp.int32, (N_PER, EPG), 0)

    A = []
    inv_deg = []
    for p in range(G_PER):
        off = (g0 + p) * N_PER
        oh_dst_t = (rows == (dst_ref[p] - off)).astype(jnp.bfloat16)
        oh_src_t = (rows == (src_ref[p] - off)).astype(jnp.bfloat16)
        a = jax.lax.dot_general(
            oh_dst_t, oh_src_t, (((1,), (1,)), ((), ())),
            preferred_element_type=jnp.float32)
        A.append(a)
        inv_deg.append(1.0 / jnp.maximum(jnp.sum(a, axis=1, keepdims=True),
                                         1.0))

    x = [jnp.dot(h_ref[p], W_emb_ref[...],
                 preferred_element_type=jnp.float32) + b_emb_ref[...]
         for p in range(G_PER)]

    bn_scale = bn_gamma_ref[...] * (1.0 / jnp.sqrt(1.0 + 1e-5))
    for i in range(4):
        agg = [jnp.dot(A[p], x[p], preferred_element_type=jnp.float32)
               * inv_deg[p] for p in range(G_PER)]
        bundle = [jnp.dot(jnp.concatenate([x[p], agg[p]], axis=1),
                          sage_W_ref[i], preferred_element_type=jnp.float32)
                  + sage_b_ref[i:i + 1, :] for p in range(G_PER)]
        for p in range(G_PER):
            ssq = jnp.sum(bundle[p] * bundle[p], axis=1, keepdims=True)
            b = bundle[p] * jax.lax.rsqrt(jnp.maximum(ssq, 1e-24))
            xx = jnp.maximum(b, 0.0)
            xx = bn_scale[i:i + 1, :] * xx + bn_beta_ref[i:i + 1, :]
            x[p] = x[p] + xx

    hg = jnp.concatenate([jnp.mean(x[p], axis=0, keepdims=True)
                          for p in range(G_PER)], axis=0)
    y = jnp.maximum(jnp.dot(hg, mlp_W0_ref[...],
                            preferred_element_type=jnp.float32)
                    + mlp_b0_ref[...], 0.0)
    y = jnp.maximum(jnp.dot(y, mlp_W1_ref[...],
                            preferred_element_type=jnp.float32)
                    + mlp_b1_ref[...], 0.0)
    z = jnp.sum(y * mlp_W2_ref[...], axis=1, keepdims=True) + mlp_b2_ref[...]
    out_ref[...] = jax.nn.sigmoid(z).reshape(1, G_PER, 1)


def kernel(h, edge_index, e, W_emb, b_emb, sage_W, sage_b, bn_gamma, bn_beta,
           dec1_W0, dec1_b0, dec1_W1, dec1_b1, dec1_W2, dec1_b2, dec2_W,
           dec2_b, mlp_W0, mlp_b0, mlp_W1, mlp_b1, mlp_W2, mlp_b2):
    h3 = h.reshape(N_GRAPHS, N_PER, HID)
    srcm = edge_index[0].reshape(N_GRAPHS, 1, EPG)
    dstm = edge_index[1].reshape(N_GRAPHS, 1, EPG)

    full = lambda *shape: pl.BlockSpec(shape, lambda g: (0,) * len(shape))
    grid_spec = pl.GridSpec(
        grid=(N_GRAPHS // G_PER,),
        in_specs=[
            pl.BlockSpec((G_PER, N_PER, HID), lambda g: (g, 0, 0)),
            pl.BlockSpec((G_PER, 1, EPG), lambda g: (g, 0, 0)),
            pl.BlockSpec((G_PER, 1, EPG), lambda g: (g, 0, 0)),
            full(HID, HID),
            full(1, HID),
            full(4, 2 * HID, HID),
            full(4, HID),
            full(4, HID),
            full(4, HID),
            full(HID, HID // 2),
            full(1, HID // 2),
            full(HID // 2, HID // 4),
            full(1, HID // 4),
            full(1, HID // 4),
            full(1, 1),
        ],
        out_specs=pl.BlockSpec((1, G_PER, 1), lambda g: (g, 0, 0)),
    )
    out = pl.pallas_call(
        _sage_body,
        grid_spec=grid_spec,
        out_shape=jax.ShapeDtypeStruct((N_GRAPHS // G_PER, G_PER, 1),
                                       jnp.float32),
    )(h3, srcm, dstm, W_emb, b_emb.reshape(1, HID), sage_W, sage_b,
      bn_gamma, bn_beta, mlp_W0, mlp_b0.reshape(1, HID // 2), mlp_W1,
      mlp_b1.reshape(1, HID // 4), mlp_W2.reshape(1, HID // 4),
      mlp_b2.reshape(1, 1))
    return out.reshape(N_GRAPHS)


# speedup vs baseline: 88.8396x; 2.2649x over previous
import jax
import jax.numpy as jnp
from jax.experimental import pallas as pl

N_GRAPHS = 100
N_PER = 100
EPG = 1600
HID = 128
G_PER = 20


def _sage_body(h_ref, ei_ref, W_emb_ref, b_emb_ref, sage_W_ref,
               sage_b_ref, bn_gamma_ref, bn_beta_ref, mlp_W0_ref, mlp_b0_ref,
               mlp_W1_ref, mlp_b1_ref, mlp_W2_ref, mlp_b2_ref, out_ref):
    g0 = pl.program_id(0) * G_PER
    rows = jax.lax.broadcasted_iota(jnp.int32, (N_PER, EPG), 0)

    A = []
    inv_deg = []
    for p in range(G_PER):
        off = (g0 + p) * N_PER
        src = ei_ref[0:1, pl.ds(EPG * p, EPG)] - off
        dst = ei_ref[1:2, pl.ds(EPG * p, EPG)] - off
        oh_dst_t = (rows == dst).astype(jnp.bfloat16)
        oh_src_t = (rows == src).astype(jnp.bfloat16)
        a = jax.lax.dot_general(
            oh_dst_t, oh_src_t, (((1,), (1,)), ((), ())),
            preferred_element_type=jnp.float32)
        A.append(a)
        inv_deg.append(1.0 / jnp.maximum(jnp.sum(a, axis=1, keepdims=True),
                                         1.0))

    x_all = (jnp.dot(h_ref[...], W_emb_ref[...],
                     preferred_element_type=jnp.float32) + b_emb_ref[...])
    x = [jax.lax.slice(x_all, (N_PER * p, 0), (N_PER * (p + 1), HID))
         for p in range(G_PER)]

    bn_scale = bn_gamma_ref[...] * (1.0 / jnp.sqrt(1.0 + 1e-5))
    for i in range(4):
        agg = [jnp.dot(A[p], x[p], preferred_element_type=jnp.float32)
               * inv_deg[p] for p in range(G_PER)]
        bundle = [jnp.dot(jnp.concatenate([x[p], agg[p]], axis=1),
                          sage_W_ref[i], preferred_element_type=jnp.float32)
                  + sage_b_ref[i:i + 1, :] for p in range(G_PER)]
        for p in range(G_PER):
            ssq = jnp.sum(bundle[p] * bundle[p], axis=1, keepdims=True)
            b = bundle[p] * jax.lax.rsqrt(jnp.maximum(ssq, 1e-24))
            xx = jnp.maximum(b, 0.0)
            xx = bn_scale[i:i + 1, :] * xx + bn_beta_ref[i:i + 1, :]
            x[p] = x[p] + xx

    hg = jnp.concatenate([jnp.mean(x[p], axis=0, keepdims=True)
                          for p in range(G_PER)], axis=0)
    y = jnp.maximum(jnp.dot(hg, mlp_W0_ref[...],
                            preferred_element_type=jnp.float32)
                    + mlp_b0_ref[...], 0.0)
    y = jnp.maximum(jnp.dot(y, mlp_W1_ref[...],
                            preferred_element_type=jnp.float32)
                    + mlp_b1_ref[...], 0.0)
    z = jnp.sum(y * mlp_W2_ref[...], axis=1, keepdims=True) + mlp_b2_ref[...]
    out_ref[...] = jax.nn.sigmoid(z).reshape(1, G_PER, 1)


def kernel(h, edge_index, e, W_emb, b_emb, sage_W, sage_b, bn_gamma, bn_beta,
           dec1_W0, dec1_b0, dec1_W1, dec1_b1, dec1_W2, dec1_b2, dec2_W,
           dec2_b, mlp_W0, mlp_b0, mlp_W1, mlp_b1, mlp_W2, mlp_b2):
    full = lambda *shape: pl.BlockSpec(shape, lambda g: (0,) * len(shape))
    grid_spec = pl.GridSpec(
        grid=(N_GRAPHS // G_PER,),
        in_specs=[
            pl.BlockSpec((G_PER * N_PER, HID), lambda g: (g, 0)),
            pl.BlockSpec((2, G_PER * EPG), lambda g: (0, g)),
            full(HID, HID),
            full(1, HID),
            full(4, 2 * HID, HID),
            full(4, HID),
            full(4, HID),
            full(4, HID),
            full(HID, HID // 2),
            full(1, HID // 2),
            full(HID // 2, HID // 4),
            full(1, HID // 4),
            full(1, HID // 4),
            full(1, 1),
        ],
        out_specs=pl.BlockSpec((1, G_PER, 1), lambda g: (g, 0, 0)),
    )
    out = pl.pallas_call(
        _sage_body,
        grid_spec=grid_spec,
        out_shape=jax.ShapeDtypeStruct((N_GRAPHS // G_PER, G_PER, 1),
                                       jnp.float32),
    )(h, edge_index, W_emb, b_emb.reshape(1, HID), sage_W, sage_b,
      bn_gamma, bn_beta, mlp_W0, mlp_b0.reshape(1, HID // 2), mlp_W1,
      mlp_b1.reshape(1, HID // 4), mlp_W2.reshape(1, HID // 4),
      mlp_b2.reshape(1, 1))
    return out.reshape(N_GRAPHS)


# speedup vs baseline: 89.8762x; 1.0117x over previous
import jax
import jax.numpy as jnp
from jax.experimental import pallas as pl

N_GRAPHS = 100
N_PER = 100
EPG = 1600
HID = 128
G_PER = 50


def _sage_body(h_ref, ei_ref, W_emb_ref, b_emb_ref, sage_W_ref,
               sage_b_ref, bn_gamma_ref, bn_beta_ref, mlp_W0_ref, mlp_b0_ref,
               mlp_W1_ref, mlp_b1_ref, mlp_W2_ref, mlp_b2_ref, out_ref):
    g0 = pl.program_id(0) * G_PER
    rows = jax.lax.broadcasted_iota(jnp.int32, (N_PER, EPG), 0)

    A = []
    inv_deg = []
    for p in range(G_PER):
        off = (g0 + p) * N_PER
        src = ei_ref[0:1, pl.ds(EPG * p, EPG)] - off
        dst = ei_ref[1:2, pl.ds(EPG * p, EPG)] - off
        oh_dst_t = (rows == dst).astype(jnp.bfloat16)
        oh_src_t = (rows == src).astype(jnp.bfloat16)
        a = jax.lax.dot_general(
            oh_dst_t, oh_src_t, (((1,), (1,)), ((), ())),
            preferred_element_type=jnp.float32)
        A.append(a)
        inv_deg.append(1.0 / jnp.maximum(jnp.sum(a, axis=1, keepdims=True),
                                         1.0))

    x_all = (jnp.dot(h_ref[...], W_emb_ref[...],
                     preferred_element_type=jnp.float32) + b_emb_ref[...])
    x = [jax.lax.slice(x_all, (N_PER * p, 0), (N_PER * (p + 1), HID))
         for p in range(G_PER)]

    bn_scale = bn_gamma_ref[...] * (1.0 / jnp.sqrt(1.0 + 1e-5))
    for i in range(4):
        agg = [jnp.dot(A[p], x[p], preferred_element_type=jnp.float32)
               * inv_deg[p] for p in range(G_PER)]
        bundle = [jnp.dot(jnp.concatenate([x[p], agg[p]], axis=1),
                          sage_W_ref[i], preferred_element_type=jnp.float32)
                  + sage_b_ref[i:i + 1, :] for p in range(G_PER)]
        for p in range(G_PER):
            ssq = jnp.sum(bundle[p] * bundle[p], axis=1, keepdims=True)
            b = bundle[p] * jax.lax.rsqrt(jnp.maximum(ssq, 1e-24))
            xx = jnp.maximum(b, 0.0)
            xx = bn_scale[i:i + 1, :] * xx + bn_beta_ref[i:i + 1, :]
            x[p] = x[p] + xx

    hg = jnp.concatenate([jnp.mean(x[p], axis=0, keepdims=True)
                          for p in range(G_PER)], axis=0)
    y = jnp.maximum(jnp.dot(hg, mlp_W0_ref[...],
                            preferred_element_type=jnp.float32)
                    + mlp_b0_ref[...], 0.0)
    y = jnp.maximum(jnp.dot(y, mlp_W1_ref[...],
                            preferred_element_type=jnp.float32)
                    + mlp_b1_ref[...], 0.0)
    z = jnp.sum(y * mlp_W2_ref[...], axis=1, keepdims=True) + mlp_b2_ref[...]
    out_ref[...] = jax.nn.sigmoid(z).reshape(1, G_PER, 1)


def kernel(h, edge_index, e, W_emb, b_emb, sage_W, sage_b, bn_gamma, bn_beta,
           dec1_W0, dec1_b0, dec1_W1, dec1_b1, dec1_W2, dec1_b2, dec2_W,
           dec2_b, mlp_W0, mlp_b0, mlp_W1, mlp_b1, mlp_W2, mlp_b2):
    full = lambda *shape: pl.BlockSpec(shape, lambda g: (0,) * len(shape))
    grid_spec = pl.GridSpec(
        grid=(N_GRAPHS // G_PER,),
        in_specs=[
            pl.BlockSpec((G_PER * N_PER, HID), lambda g: (g, 0)),
            pl.BlockSpec((2, G_PER * EPG), lambda g: (0, g)),
            full(HID, HID),
            full(1, HID),
            full(4, 2 * HID, HID),
            full(4, HID),
            full(4, HID),
            full(4, HID),
            full(HID, HID // 2),
            full(1, HID // 2),
            full(HID // 2, HID // 4),
            full(1, HID // 4),
            full(1, HID // 4),
            full(1, 1),
        ],
        out_specs=pl.BlockSpec((1, G_PER, 1), lambda g: (g, 0, 0)),
    )
    out = pl.pallas_call(
        _sage_body,
        grid_spec=grid_spec,
        out_shape=jax.ShapeDtypeStruct((N_GRAPHS // G_PER, G_PER, 1),
                                       jnp.float32),
    )(h, edge_index, W_emb, b_emb.reshape(1, HID), sage_W, sage_b,
      bn_gamma, bn_beta, mlp_W0, mlp_b0.reshape(1, HID // 2), mlp_W1,
      mlp_b1.reshape(1, HID // 4), mlp_W2.reshape(1, HID // 4),
      mlp_b2.reshape(1, 1))
    return out.reshape(N_GRAPHS)
